# baseline (speedup 1.0000x reference)
import functools

import jax
import jax.numpy as jnp
from jax import lax
from jax.experimental import pallas as pl
from jax.experimental.pallas import tpu as pltpu
from jax.experimental.pallas import tpu_sc as plsc

_B = 16384
_D = 32
_NC = 2
_NS = 16
_NW = _NC * _NS
_BPW = _B // _NW
_L = 16


def _tec_body(table_hbm, c1_hbm, c2_hbm, out_hbm,
              idx1_v, idx2_v, rows1_v, rows2_v, out_v, sem1, sem2):
    wid = lax.axis_index("s") * _NC + lax.axis_index("c")
    base = wid * _BPW

    pltpu.sync_copy(c1_hbm.at[pl.ds(base, _BPW)], idx1_v)
    pltpu.sync_copy(c2_hbm.at[pl.ds(base, _BPW)], idx2_v)
    g1 = pltpu.async_copy(table_hbm.at[idx1_v], rows1_v, sem1)
    g2 = pltpu.async_copy(table_hbm.at[idx2_v], rows2_v, sem2)
    g1.wait()
    g2.wait()

    lane = lax.iota(jnp.int32, _L)

    def blk_body(blk, carry):
        row_idx = blk * _L + lane
        acc = jnp.zeros((_L,), jnp.float32)
        for j in range(_D):
            jv = jnp.full((_L,), j, jnp.int32)
            a = plsc.load_gather(rows1_v, [row_idx, jv])
            b = plsc.load_gather(rows2_v, [row_idx, jv])
            acc = acc + a * b
        out_v[pl.ds(blk * _L, _L)] = acc
        return carry

    lax.fori_loop(0, _BPW // _L, blk_body, 0)

    pltpu.sync_copy(out_v, out_hbm.at[pl.ds(base, _BPW)])


_gather_dot = functools.partial(
    pl.kernel,
    mesh=plsc.VectorSubcoreMesh(core_axis_name="c", subcore_axis_name="s"),
    out_type=jax.ShapeDtypeStruct((_B,), jnp.float32),
    compiler_params=pltpu.CompilerParams(
        needs_layout_passes=False, use_tc_tiling_on_sc=False
    ),
    scratch_types=[
        pltpu.VMEM((_BPW,), jnp.int32),
        pltpu.VMEM((_BPW,), jnp.int32),
        pltpu.VMEM((_BPW, _D), jnp.float32),
        pltpu.VMEM((_BPW, _D), jnp.float32),
        pltpu.VMEM((_BPW,), jnp.float32),
        pltpu.SemaphoreType.DMA,
        pltpu.SemaphoreType.DMA,
    ],
)(_tec_body)


@jax.jit
def kernel(champ1, champ2, table):
    c1 = champ1.astype(jnp.int32)
    c2 = champ2.astype(jnp.int32)
    dot = _gather_dot(table, c1, c2)
    return dot.reshape(-1, 1, 1)


# speedup vs baseline: 1.1442x; 1.1442x over previous
import functools

import jax
import jax.numpy as jnp
from jax import lax
from jax.experimental import pallas as pl
from jax.experimental.pallas import tpu as pltpu
from jax.experimental.pallas import tpu_sc as plsc

_B = 16384
_D = 32
_V = 1_000_000
_NC = 2
_NS = 16
_NW = _NC * _NS
_BPW = _B // _NW
_L = 16

_CB = 2048
_NBLK = (_V + _CB - 1) // _CB
_OUTR = _V * _D // 128


def _detile_body(in_ref, out_ref, scr):
    scr[...] = in_ref[...].T
    for q in range(4):
        out_ref[:, q * 32:(q + 1) * 32] = scr[pl.Slice(q, _CB // 4, 4), :]


_detile = pl.pallas_call(
    _detile_body,
    grid=(_NBLK,),
    in_specs=[pl.BlockSpec((_D, _CB), lambda i: (0, i))],
    out_specs=pl.BlockSpec((_CB * _D // 128, 128), lambda i: (i, 0)),
    out_shape=jax.ShapeDtypeStruct((_OUTR, 128), jnp.float32),
    scratch_shapes=[pltpu.VMEM((_CB, _D), jnp.float32)],
)


def _tec_body(rows_hbm, c1_hbm, c2_hbm, out_hbm,
              idx1_v, idx2_v, rows1_v, rows2_v, out_v, sem1, sem2):
    wid = lax.axis_index("s") * _NC + lax.axis_index("c")
    base = wid * _BPW

    pltpu.sync_copy(c1_hbm.at[pl.ds(base, _BPW)], idx1_v)
    pltpu.sync_copy(c2_hbm.at[pl.ds(base, _BPW)], idx2_v)
    g1 = pltpu.async_copy(rows_hbm.at[idx1_v], rows1_v, sem1)
    g2 = pltpu.async_copy(rows_hbm.at[idx2_v], rows2_v, sem2)
    g1.wait()
    g2.wait()

    lane = lax.iota(jnp.int32, _L)

    def blk_body(blk, carry):
        row_idx = blk * _L + lane
        acc = jnp.zeros((_L,), jnp.float32)
        for j in range(_D):
            jv = jnp.full((_L,), j, jnp.int32)
            a = plsc.load_gather(rows1_v, [row_idx, jv])
            b = plsc.load_gather(rows2_v, [row_idx, jv])
            acc = acc + a * b
        out_v[pl.ds(blk * _L, _L)] = acc
        return carry

    lax.fori_loop(0, _BPW // _L, blk_body, 0)

    pltpu.sync_copy(out_v, out_hbm.at[pl.ds(base, _BPW)])


_gather_dot = functools.partial(
    pl.kernel,
    mesh=plsc.VectorSubcoreMesh(core_axis_name="c", subcore_axis_name="s"),
    out_type=jax.ShapeDtypeStruct((_B,), jnp.float32),
    compiler_params=pltpu.CompilerParams(
        needs_layout_passes=False, use_tc_tiling_on_sc=False
    ),
    scratch_types=[
        pltpu.VMEM((_BPW,), jnp.int32),
        pltpu.VMEM((_BPW,), jnp.int32),
        pltpu.VMEM((_BPW, _D), jnp.float32),
        pltpu.VMEM((_BPW, _D), jnp.float32),
        pltpu.VMEM((_BPW,), jnp.float32),
        pltpu.SemaphoreType.DMA,
        pltpu.SemaphoreType.DMA,
    ],
)(_tec_body)


@jax.jit
def kernel(champ1, champ2, table):
    c1 = champ1.astype(jnp.int32)
    c2 = champ2.astype(jnp.int32)
    t2 = jnp.swapaxes(table, 0, 1)
    rows = _detile(t2).reshape(_V, _D)
    dot = _gather_dot(rows, c1, c2)
    return dot.reshape(-1, 1, 1)
